# Initial kernel scaffold; baseline (speedup 1.0000x reference)
import functools

import jax
import jax.numpy as jnp
from jax import lax
from jax.experimental import pallas as pl
from jax.experimental.pallas import tpu as pltpu
from jax.experimental.pallas import tpu_sc as plsc

N_NODES = 10000
N_EDGES = 320000
D_FEAT = 128

NC = 2
NS = 16
NW = NC * NS
E_PER_W = N_EDGES // NW
CHUNK = 80
NCHUNK = E_PER_W // CHUNK
ROWS_PER_TILE = N_NODES // NS
ZROWS = 125

_MESH = plsc.VectorSubcoreMesh(core_axis_name="c", subcore_axis_name="s")


def _fill_zeros(ref, nrows, width):
    def body(i, _):
        for j in range(width // 16):
            ref[i, pl.ds(j * 16, 16)] = jnp.zeros((16,), jnp.float32)
        return 0
    lax.fori_loop(0, nrows, body, 0)


def _deg_body(src_hbm, dst_hbm, out_hbm, idx_v, ones_v, z_v, dsrc_sh, ddst_sh):
    core = lax.axis_index("c")
    sid = lax.axis_index("s")
    wid = sid * NC + core

    _fill_zeros(z_v, ROWS_PER_TILE, 16)
    def ones_body(i, _):
        ones_v[i, pl.ds(0, 16)] = jnp.ones((16,), jnp.float32)
        return 0
    lax.fori_loop(0, CHUNK, ones_body, 0)

    row0 = sid * ROWS_PER_TILE
    pltpu.sync_copy(z_v, dsrc_sh.at[pl.ds(row0, ROWS_PER_TILE)])
    pltpu.sync_copy(z_v, ddst_sh.at[pl.ds(row0, ROWS_PER_TILE)])
    plsc.subcore_barrier()

    def body(ci, _):
        base = wid * E_PER_W + ci * CHUNK
        pltpu.sync_copy(src_hbm.at[pl.ds(base, CHUNK)], idx_v)
        pltpu.sync_copy(ones_v, dsrc_sh.at[idx_v], add=True)
        pltpu.sync_copy(dst_hbm.at[pl.ds(base, CHUNK)], idx_v)
        pltpu.sync_copy(ones_v, ddst_sh.at[idx_v], add=True)
        return 0
    lax.fori_loop(0, NCHUNK, body, 0)

    plsc.subcore_barrier()
    pltpu.sync_copy(dsrc_sh.at[pl.ds(row0, ROWS_PER_TILE)],
                    out_hbm.at[core, 0, pl.ds(row0, ROWS_PER_TILE)])
    pltpu.sync_copy(ddst_sh.at[pl.ds(row0, ROWS_PER_TILE)],
                    out_hbm.at[core, 1, pl.ds(row0, ROWS_PER_TILE)])


_deg_call = functools.partial(
    pl.kernel,
    out_type=jax.ShapeDtypeStruct((NC, 2, N_NODES, 16), jnp.float32),
    mesh=_MESH,
    scratch_types=[
        pltpu.VMEM((CHUNK,), jnp.int32),
        pltpu.VMEM((CHUNK, 16), jnp.float32),
        pltpu.VMEM((ROWS_PER_TILE, 16), jnp.float32),
        pltpu.VMEM_SHARED((N_NODES, 16), jnp.float32),
        pltpu.VMEM_SHARED((N_NODES, 16), jnp.float32),
    ],
)(_deg_body)


def _gather_body(m_hbm, src_hbm, dst_hbm, out_hbm,
                 sidx_v, didx_v, rows_v, z_v, agg_sh, sem):
    core = lax.axis_index("c")
    sid = lax.axis_index("s")
    wid = sid * NC + core

    _fill_zeros(z_v, ZROWS, D_FEAT)
    for r in range(ROWS_PER_TILE // ZROWS):
        pltpu.sync_copy(z_v, agg_sh.at[pl.ds((sid * 5 + r) * ZROWS, ZROWS)])
    plsc.subcore_barrier()

    def body(ci, _):
        base = wid * E_PER_W + ci * CHUNK
        pltpu.sync_copy(src_hbm.at[pl.ds(base, CHUNK)], sidx_v)
        pltpu.async_copy(m_hbm.at[sidx_v], rows_v, sem).wait()
        pltpu.sync_copy(dst_hbm.at[pl.ds(base, CHUNK)], didx_v)
        pltpu.sync_copy(rows_v, agg_sh.at[didx_v], add=True)
        return 0
    lax.fori_loop(0, NCHUNK, body, 0)

    plsc.subcore_barrier()
    for r in range(ROWS_PER_TILE // ZROWS):
        rs = (sid * 5 + r) * ZROWS
        pltpu.sync_copy(agg_sh.at[pl.ds(rs, ZROWS)],
                        out_hbm.at[core, pl.ds(rs, ZROWS)])


_gather_call = functools.partial(
    pl.kernel,
    out_type=jax.ShapeDtypeStruct((NC, N_NODES, D_FEAT), jnp.float32),
    mesh=_MESH,
    scratch_types=[
        pltpu.VMEM((CHUNK,), jnp.int32),
        pltpu.VMEM((CHUNK,), jnp.int32),
        pltpu.VMEM((CHUNK, D_FEAT), jnp.float32),
        pltpu.VMEM((ZROWS, D_FEAT), jnp.float32),
        pltpu.VMEM_SHARED((N_NODES, D_FEAT), jnp.float32),
        pltpu.SemaphoreType.DMA,
    ],
)(_gather_body)


def _norm_m0_body(degp_ref, x_ref, ns_ref, nd_ref, m0_ref):
    dsrc = degp_ref[0, 0] + degp_ref[1, 0]
    ddst = degp_ref[0, 1] + degp_ref[1, 1]
    ns = 1.0 / jnp.sqrt(jnp.maximum(dsrc[:, 0:1], 1.0))
    nd = 1.0 / jnp.sqrt(jnp.maximum(ddst[:, 0:1], 1.0))
    ns_ref[...] = ns
    nd_ref[...] = nd
    m0_ref[...] = x_ref[...] * ns


def _norm_m0(degp, x):
    return pl.pallas_call(
        _norm_m0_body,
        out_shape=[
            jax.ShapeDtypeStruct((N_NODES, 1), jnp.float32),
            jax.ShapeDtypeStruct((N_NODES, 1), jnp.float32),
            jax.ShapeDtypeStruct((N_NODES, D_FEAT), jnp.float32),
        ],
    )(degp, x)


_TC_ROWS = 1250


def _layer_tc_body(p_ref, nd_ref, ns_ref, w_ref, b_ref, o_ref, *, last):
    agg = (p_ref[0] + p_ref[1]) * nd_ref[...]
    h = jnp.dot(agg, w_ref[...], preferred_element_type=jnp.float32)
    h = jnp.maximum(h + b_ref[...], 0.0)
    o_ref[...] = h if last else h * ns_ref[...]


def _layer_tc(parts, nd, ns, w, b, last):
    grid = (N_NODES // _TC_ROWS,)
    return pl.pallas_call(
        functools.partial(_layer_tc_body, last=last),
        grid=grid,
        in_specs=[
            pl.BlockSpec((NC, _TC_ROWS, D_FEAT), lambda i: (0, i, 0)),
            pl.BlockSpec((_TC_ROWS, 1), lambda i: (i, 0)),
            pl.BlockSpec((_TC_ROWS, 1), lambda i: (i, 0)),
            pl.BlockSpec((D_FEAT, D_FEAT), lambda i: (0, 0)),
            pl.BlockSpec((1, D_FEAT), lambda i: (0, 0)),
        ],
        out_specs=pl.BlockSpec((_TC_ROWS, D_FEAT), lambda i: (i, 0)),
        out_shape=jax.ShapeDtypeStruct((N_NODES, D_FEAT), jnp.float32),
    )(parts, nd, ns, w, b)


def kernel(x, edge_index, W0, b0, W1, b1, W2, b2, W3, b3, W4, b4):
    src = edge_index[0].astype(jnp.int32)
    dst = edge_index[1].astype(jnp.int32)
    Ws = [W0, W1, W2, W3, W4]
    bs = [b0, b1, b2, b3, b4]

    degp = _deg_call(src, dst)
    ns, nd, m = _norm_m0(degp, x)
    for i in range(5):
        parts = _gather_call(m, src, dst)
        m = _layer_tc(parts, nd, ns, Ws[i], bs[i].reshape(1, D_FEAT),
                      last=(i == 4))
    return m


# speedup vs baseline: 4.5990x; 4.5990x over previous
import functools

import jax
import jax.numpy as jnp
from jax import lax
from jax.experimental import pallas as pl
from jax.experimental.pallas import tpu as pltpu
from jax.experimental.pallas import tpu_sc as plsc

N_NODES = 10000
N_EDGES = 320000
D_FEAT = 128

NC = 2
NS = 16
NW = NC * NS
E_PER_W = N_EDGES // NW
CHUNK = 80
NCHUNK = E_PER_W // CHUNK
N_PAD = 10240
ROWS_PER_TILE = N_PAD // NS
ZROWS = 128

_MESH = plsc.VectorSubcoreMesh(core_axis_name="c", subcore_axis_name="s")


def _fill_const(ref, nrows, width, val):
    def body(i, _):
        for j in range(width // 16):
            ref[i, pl.ds(j * 16, 16)] = jnp.full((16,), val, jnp.float32)
        return 0
    lax.fori_loop(0, nrows, body, 0)


def _zero_agg(z_v, agg_sh, sid):
    for r in range(ROWS_PER_TILE // ZROWS):
        pltpu.sync_copy(z_v, agg_sh.at[pl.ds(sid * ROWS_PER_TILE + r * ZROWS,
                                             ZROWS)])


def _write_out(z_v, agg_sh, out_slot, sid):
    for r in range(ROWS_PER_TILE // ZROWS):
        rs = sid * ROWS_PER_TILE + r * ZROWS
        pltpu.sync_copy(agg_sh.at[pl.ds(rs, ZROWS)], z_v)
        pltpu.sync_copy(z_v, out_slot.at[pl.ds(rs, ZROWS)])


def _deg_body(src_hbm, dst_hbm, out_hbm, idx_v, ones_v, z_v, hist_sh):
    core = lax.axis_index("c")
    sid = lax.axis_index("s")
    wid = sid * NC + core

    _fill_const(ones_v, CHUNK, D_FEAT, 1.0)

    for phase, edge_hbm in enumerate((src_hbm, dst_hbm)):
        _fill_const(z_v, ZROWS, D_FEAT, 0.0)
        _zero_agg(z_v, hist_sh, sid)
        plsc.subcore_barrier()

        def body(ci, _):
            base = wid * E_PER_W + ci * CHUNK
            pltpu.sync_copy(edge_hbm.at[pl.ds(base, CHUNK)], idx_v)
            pltpu.sync_copy(ones_v, hist_sh.at[idx_v], add=True)
            return 0
        lax.fori_loop(0, NCHUNK, body, 0)

        plsc.subcore_barrier()
        _write_out(z_v, hist_sh, out_hbm.at[core, phase], sid)
        plsc.subcore_barrier()


_deg_call = functools.partial(
    pl.kernel,
    out_type=jax.ShapeDtypeStruct((NC, 2, N_PAD, D_FEAT), jnp.float32),
    mesh=_MESH,
    scratch_types=[
        pltpu.VMEM((CHUNK,), jnp.int32),
        pltpu.VMEM((CHUNK, D_FEAT), jnp.float32),
        pltpu.VMEM((ZROWS, D_FEAT), jnp.float32),
        pltpu.VMEM_SHARED((N_PAD, D_FEAT), jnp.float32),
    ],
)(_deg_body)


def _gather_body(m_hbm, src_hbm, dst_hbm, out_hbm,
                 sidx_v, didx_v, rows_v, z_v, agg_sh, sem):
    core = lax.axis_index("c")
    sid = lax.axis_index("s")
    wid = sid * NC + core

    _fill_const(z_v, ZROWS, D_FEAT, 0.0)
    _zero_agg(z_v, agg_sh, sid)
    plsc.subcore_barrier()

    def body(ci, _):
        base = wid * E_PER_W + ci * CHUNK
        pltpu.sync_copy(src_hbm.at[pl.ds(base, CHUNK)], sidx_v)
        pltpu.async_copy(m_hbm.at[sidx_v], rows_v, sem).wait()
        pltpu.sync_copy(dst_hbm.at[pl.ds(base, CHUNK)], didx_v)
        pltpu.sync_copy(rows_v, agg_sh.at[didx_v], add=True)
        return 0
    lax.fori_loop(0, NCHUNK, body, 0)

    plsc.subcore_barrier()
    _write_out(z_v, agg_sh, out_hbm.at[core], sid)


_gather_call = functools.partial(
    pl.kernel,
    out_type=jax.ShapeDtypeStruct((NC, N_PAD, D_FEAT), jnp.float32),
    mesh=_MESH,
    scratch_types=[
        pltpu.VMEM((CHUNK,), jnp.int32),
        pltpu.VMEM((CHUNK,), jnp.int32),
        pltpu.VMEM((CHUNK, D_FEAT), jnp.float32),
        pltpu.VMEM((ZROWS, D_FEAT), jnp.float32),
        pltpu.VMEM_SHARED((N_PAD, D_FEAT), jnp.float32),
        pltpu.SemaphoreType.DMA,
    ],
)(_gather_body)


def _norm_m0_body(degp_ref, x_ref, ns_ref, nd_ref, m0_ref):
    dsrc = degp_ref[0, 0] + degp_ref[1, 0]
    ddst = degp_ref[0, 1] + degp_ref[1, 1]
    ns = 1.0 / jnp.sqrt(jnp.maximum(dsrc[:, 0:1], 1.0))
    nd = 1.0 / jnp.sqrt(jnp.maximum(ddst[:, 0:1], 1.0))
    ns_ref[...] = ns
    nd_ref[...] = nd
    m0_ref[...] = x_ref[...] * ns[0:N_NODES, :]


def _norm_m0(degp, x):
    return pl.pallas_call(
        _norm_m0_body,
        out_shape=[
            jax.ShapeDtypeStruct((N_PAD, 1), jnp.float32),
            jax.ShapeDtypeStruct((N_PAD, 1), jnp.float32),
            jax.ShapeDtypeStruct((N_NODES, D_FEAT), jnp.float32),
        ],
    )(degp, x)


_TC_ROWS = 1000


def _layer_tc_body(p_ref, nd_ref, ns_ref, w_ref, b_ref, o_ref, *, last):
    agg = (p_ref[0] + p_ref[1]) * nd_ref[...]
    h = jnp.dot(agg, w_ref[...], preferred_element_type=jnp.float32)
    h = jnp.maximum(h + b_ref[...], 0.0)
    o_ref[...] = h if last else h * ns_ref[...]


def _layer_tc(parts, nd, ns, w, b, last):
    grid = (N_NODES // _TC_ROWS,)
    return pl.pallas_call(
        functools.partial(_layer_tc_body, last=last),
        grid=grid,
        in_specs=[
            pl.BlockSpec((NC, _TC_ROWS, D_FEAT), lambda i: (0, i, 0)),
            pl.BlockSpec((_TC_ROWS, 1), lambda i: (i, 0)),
            pl.BlockSpec((_TC_ROWS, 1), lambda i: (i, 0)),
            pl.BlockSpec((D_FEAT, D_FEAT), lambda i: (0, 0)),
            pl.BlockSpec((1, D_FEAT), lambda i: (0, 0)),
        ],
        out_specs=pl.BlockSpec((_TC_ROWS, D_FEAT), lambda i: (i, 0)),
        out_shape=jax.ShapeDtypeStruct((N_NODES, D_FEAT), jnp.float32),
    )(parts, nd, ns, w, b)


def kernel(x, edge_index, W0, b0, W1, b1, W2, b2, W3, b3, W4, b4):
    src = edge_index[0].astype(jnp.int32)
    dst = edge_index[1].astype(jnp.int32)
    Ws = [W0, W1, W2, W3, W4]
    bs = [b0, b1, b2, b3, b4]

    degp = _deg_call(src, dst)
    ns, nd, m = _norm_m0(degp, x)
    for i in range(5):
        parts = _gather_call(m, src, dst)
        m = _layer_tc(parts, nd, ns, Ws[i], bs[i].reshape(1, D_FEAT),
                      last=(i == 4))
    return m
